# baseline (speedup 1.0000x reference)
import functools

import jax
import jax.numpy as jnp
from jax import lax
from jax.experimental import pallas as pl
from jax.experimental.pallas import tpu as pltpu
from jax.experimental.pallas import tpu_sc as plsc

N_NODES = 10000
N_EDGES = 320000
D = 128
DP = 128
N_GRAPHS = 64

NC = 2
NS = 16
NW = NC * NS
CH = 128
NCHUNK = 80
EPT = NCHUNK * CH
E_PAD = NW * EPT
WB = 624
ZB = 104
NZB = WB // ZB
REM = N_NODES - NS * WB

_mesh = plsc.VectorSubcoreMesh(core_axis_name="c", subcore_axis_name="s")


def _make_deg_kernel():
    @functools.partial(
        pl.kernel,
        out_type=jax.ShapeDtypeStruct((NC, N_NODES), jnp.float32),
        mesh=_mesh,
        scratch_types=[
            pltpu.VMEM((NCHUNK, CH), jnp.int32),
            pltpu.VMEM((NCHUNK, CH), jnp.float32),
            pltpu.VMEM((N_NODES,), jnp.float32),
            pltpu.VMEM_SHARED((N_NODES,), jnp.float32),
            pltpu.SemaphoreType.DMA,
        ],
    )
    def deg_kernel(col_hbm, ew_hbm, out_hbm, col2, ew2, zero_v, acc_sh, sem):
        cid = lax.axis_index("c")
        sid = lax.axis_index("s")
        wid = cid * NS + sid
        zeros16 = jnp.zeros((16,), jnp.float32)

        pltpu.sync_copy(col_hbm.at[wid], col2)
        pltpu.sync_copy(ew_hbm.at[wid], ew2)

        @pl.when(sid == 0)
        def _zero():
            def zr(i, _):
                zero_v[pl.ds(i * 16, 16)] = zeros16
                return 0
            lax.fori_loop(0, N_NODES // 16, zr, 0)
            pltpu.sync_copy(zero_v, acc_sh)

        plsc.subcore_barrier()

        def body(ci, _):
            pltpu.async_copy(ew2.at[ci], acc_sh.at[col2.at[ci]], sem, add=True)
            return 0
        lax.fori_loop(0, NCHUNK, body, 0)

        def drain(ci, _):
            pltpu.make_async_copy(ew2.at[0], acc_sh.at[col2.at[0]], sem).wait()
            return 0
        lax.fori_loop(0, NCHUNK, drain, 0)

        plsc.subcore_barrier()

        @pl.when(sid == 0)
        def _out():
            pltpu.sync_copy(acc_sh, out_hbm.at[cid])

    return deg_kernel


NBUF = 3


def _make_msg_kernel(F):
    @functools.partial(
        pl.kernel,
        out_type=jax.ShapeDtypeStruct((NC, N_NODES, F), jnp.float32),
        mesh=_mesh,
        scratch_types=[
            pltpu.VMEM((2 * NBUF, CH), jnp.int32),
            [pltpu.VMEM((CH,), jnp.int32)] * NBUF,
            [pltpu.VMEM((CH,), jnp.float32)] * NBUF,
            [pltpu.VMEM((CH, F), jnp.float32)] * NBUF,
            pltpu.VMEM_SHARED((N_NODES, F), jnp.float32),
            [pltpu.SemaphoreType.DMA] * NBUF,
            [pltpu.SemaphoreType.DMA] * NBUF,
            [pltpu.SemaphoreType.DMA] * NBUF,
        ],
    )
    def msg_kernel(xs_hbm, row_hbm, col_hbm, ew_hbm, out_hbm,
                   colb, rowb, ewb, msgs, acc_sh, isems, gsems, ssems):
        cid = lax.axis_index("c")
        sid = lax.axis_index("s")
        wid = cid * NS + sid
        ebase = wid * EPT
        zeros16 = jnp.zeros((16,), jnp.float32)

        def zr(i, _):
            for f in range(F // 16):
                msgs[0][i, pl.ds(f * 16, 16)] = zeros16
            return 0
        lax.fori_loop(0, CH, zr, 0)

        rbase = sid * WB
        for b in range(WB // CH):
            pltpu.sync_copy(msgs[0], acc_sh.at[pl.ds(rbase + b * CH, CH)])
        zrem = WB - (WB // CH) * CH
        pltpu.sync_copy(msgs[0].at[pl.ds(0, zrem)],
                        acc_sh.at[pl.ds(rbase + (WB // CH) * CH, zrem)])

        @pl.when(sid == 0)
        def _zrem():
            pltpu.sync_copy(msgs[0].at[pl.ds(0, REM)],
                            acc_sh.at[pl.ds(NS * WB, REM)])

        plsc.subcore_barrier()

        def i_start(c, b):
            off = ebase + c * CH
            pltpu.async_copy(row_hbm.at[pl.ds(off, CH)], rowb[b], isems[b])
            pltpu.async_copy(ew_hbm.at[pl.ds(off, CH)], ewb[b], isems[b])
            pltpu.async_copy(col_hbm.at[wid, c],
                             colb.at[pl.ds(c % (2 * NBUF), 1)], isems[b])

        def i_wait(b):
            pltpu.make_async_copy(row_hbm.at[pl.ds(0, CH)], rowb[b],
                                  isems[b]).wait()
            pltpu.make_async_copy(ew_hbm.at[pl.ds(0, CH)], ewb[b],
                                  isems[b]).wait()
            pltpu.make_async_copy(col_hbm.at[0, 0],
                                  colb.at[pl.ds(0, 1)], isems[b]).wait()

        def g_start(b):
            pltpu.async_copy(xs_hbm.at[rowb[b]], msgs[b], gsems[b])

        def g_wait(b):
            pltpu.make_async_copy(xs_hbm.at[rowb[b]], msgs[b],
                                  gsems[b]).wait()

        def s_start(c, b):
            pltpu.async_copy(msgs[b], acc_sh.at[colb.at[c % (2 * NBUF)]],
                             ssems[b], add=True)

        def s_wait(b):
            pltpu.make_async_copy(msgs[b], acc_sh.at[colb.at[0]],
                                  ssems[b]).wait()

        def scale(c, b):
            def sc(j, _):
                w16 = ewb[b][pl.ds(j * 16, 16)]
                for l in range(16):
                    wl = w16[l]
                    e = j * 16 + l
                    for f in range(F // 16):
                        msgs[b][e, pl.ds(f * 16, 16)] = (
                            msgs[b][e, pl.ds(f * 16, 16)] * wl)
                return 0
            lax.fori_loop(0, CH // 16, sc, 0)

        def process(c, b):
            b1 = (b + 1) % NBUF
            b2 = (b + 2) % NBUF
            nxt = c + 1

            @pl.when(c + 2 < NCHUNK)
            def _():
                i_start(c + 2, b2)

            @pl.when(jnp.logical_and(nxt < NCHUNK, nxt >= NBUF))
            def _():
                s_wait(b1)

            @pl.when(nxt < NCHUNK)
            def _():
                i_wait(b1)
                g_start(b1)

            g_wait(b)
            scale(c, b)
            s_start(c, b)

        i_start(0, 0)
        i_start(1, 1)
        i_wait(0)
        g_start(0)

        NOUT = NCHUNK // NBUF

        def outer(i, _):
            for b in range(NBUF):
                process(i * NBUF + b, b)
            return 0
        lax.fori_loop(0, NOUT, outer, 0)
        for b in range(NCHUNK - NOUT * NBUF):
            process(NOUT * NBUF + b, b)
        for k in range(NBUF):
            s_wait((NCHUNK - NBUF + k) % NBUF)

        plsc.subcore_barrier()

        for b in range(WB // CH):
            pltpu.sync_copy(acc_sh.at[pl.ds(rbase + b * CH, CH)],
                            out_hbm.at[cid, pl.ds(rbase + b * CH, CH)])
        pltpu.sync_copy(acc_sh.at[pl.ds(rbase + (WB // CH) * CH, zrem)],
                        out_hbm.at[cid, pl.ds(rbase + (WB // CH) * CH, zrem)])

        @pl.when(sid == 0)
        def _wrem():
            pltpu.sync_copy(acc_sh.at[pl.ds(NS * WB, REM)],
                            out_hbm.at[cid, pl.ds(NS * WB, REM)])

    return msg_kernel


_deg_kernel = _make_deg_kernel()
_msg_kernel_128 = _make_msg_kernel(D)


def _mm_body(x_ref, w_ref, out_ref):
    out_ref[...] = jnp.dot(x_ref[...], w_ref[...],
                           preferred_element_type=jnp.float32)


def _prep_body(degp_ref, xw_ref, dinv_ref, xs_ref):
    deg = 1.0 + degp_ref[0] + degp_ref[1]
    dinv = jnp.where(deg > 0, lax.rsqrt(deg), 0.0)
    dinv_ref[...] = dinv
    xs_ref[...] = dinv * xw_ref[...]


def _combine_body(s_ref, xs_ref, dinv_ref, b_ref, w_ref, out_ref):
    s = s_ref[0] + s_ref[1] + xs_ref[...]
    h = jnp.maximum(dinv_ref[...] * s + b_ref[...], 0.0)
    out_ref[...] = dinv_ref[...] * jnp.dot(h, w_ref[...],
                                           preferred_element_type=jnp.float32)


def _final_body(s_ref, xs_ref, dinv_ref, b_ref, batch_ref, out_ref):
    o = dinv_ref[...] * (s_ref[0] + s_ref[1] + xs_ref[...]) + b_ref[...]
    gids = lax.broadcasted_iota(jnp.int32, (N_NODES, N_GRAPHS), 1)
    onehot = (batch_ref[...] == gids).astype(jnp.float32)
    sums = lax.dot_general(onehot, o, (((0,), (0,)), ((), ())))
    ones = jnp.ones((N_NODES, 1), jnp.float32)
    cnt = lax.dot_general(onehot, ones, (((0,), (0,)), ((), ())))
    mean = sums / jnp.maximum(cnt, 1.0)
    cols = lax.broadcasted_iota(jnp.int32, (N_GRAPHS, DP), 1)
    logits = jnp.where(cols < 10, mean, -1e30)
    m = jnp.max(logits, axis=1, keepdims=True)
    z = logits - m
    lse = jnp.log(jnp.sum(jnp.exp(z), axis=1, keepdims=True))
    out_ref[...] = z - lse


def _tc_call(body, out_shapes, *args):
    return pl.pallas_call(
        body,
        out_shape=out_shapes,
    )(*args)


def kernel(x, edge_index, edge_attr, batch, W1, b1, W2, b2, W3, b3):
    npad = E_PAD - N_EDGES
    row = jnp.pad(edge_index[0].astype(jnp.int32), (0, npad))
    col = jnp.pad(edge_index[1].astype(jnp.int32), (0, npad))
    ew = jnp.pad(edge_attr.astype(jnp.float32), (0, npad))
    col3 = col.reshape(NW, NCHUNK, CH)
    col4 = col.reshape(NW, NCHUNK, 1, CH)
    ew3 = ew.reshape(NW, NCHUNK, CH)
    batch2 = batch.astype(jnp.int32).reshape(N_NODES, 1)

    W3p = jnp.pad(W3, ((0, 0), (0, DP - W3.shape[1])))
    b3p = jnp.pad(b3, (0, DP - b3.shape[0])).reshape(1, DP)
    b1r = b1.reshape(1, D)
    b2r = b2.reshape(1, D)

    degp = _deg_kernel(col3, ew3)
    degp3 = degp.reshape(NC, N_NODES, 1)
    xw1 = _tc_call(_mm_body,
                   jax.ShapeDtypeStruct((N_NODES, D), jnp.float32),
                   x, W1)

    dinv, xs1 = _tc_call(
        _prep_body,
        [jax.ShapeDtypeStruct((N_NODES, 1), jnp.float32),
         jax.ShapeDtypeStruct((N_NODES, D), jnp.float32)],
        degp3, xw1)

    s1 = _msg_kernel_128(xs1, row, col4, ew)
    xs2 = _tc_call(
        _combine_body,
        jax.ShapeDtypeStruct((N_NODES, D), jnp.float32),
        s1, xs1, dinv, b1r, W2)

    s2 = _msg_kernel_128(xs2, row, col4, ew)
    xs3 = _tc_call(
        _combine_body,
        jax.ShapeDtypeStruct((N_NODES, DP), jnp.float32),
        s2, xs2, dinv, b2r, W3p)

    s3 = _msg_kernel_128(xs3, row, col4, ew)
    outp = _tc_call(
        _final_body,
        jax.ShapeDtypeStruct((N_GRAPHS, DP), jnp.float32),
        s3, xs3, dinv, b3p, batch2)

    return outp[:, :10]


# speedup vs baseline: 3.5645x; 3.5645x over previous
import functools

import jax
import jax.numpy as jnp
from jax import lax
from jax.experimental import pallas as pl
from jax.experimental.pallas import tpu as pltpu
from jax.experimental.pallas import tpu_sc as plsc

N_NODES = 10000
N_EDGES = 320000
D = 128
DP = 128
N_GRAPHS = 64

NC = 2
NS = 16
NW = NC * NS
EPT = N_EDGES // NW
CH = 80
NCHUNK = EPT // CH
WB = 624
ZB = 104
NZB = WB // ZB
REM = N_NODES - NS * WB

_mesh = plsc.VectorSubcoreMesh(core_axis_name="c", subcore_axis_name="s")


def _make_deg_kernel():
    @functools.partial(
        pl.kernel,
        out_type=jax.ShapeDtypeStruct((NC, N_NODES), jnp.float32),
        mesh=_mesh,
        scratch_types=[
            pltpu.VMEM((NCHUNK, CH), jnp.int32),
            pltpu.VMEM((NCHUNK, CH), jnp.float32),
            pltpu.VMEM((N_NODES,), jnp.float32),
            pltpu.VMEM_SHARED((N_NODES,), jnp.float32),
            pltpu.SemaphoreType.DMA,
        ],
    )
    def deg_kernel(col_hbm, ew_hbm, out_hbm, col2, ew2, zero_v, acc_sh, sem):
        cid = lax.axis_index("c")
        sid = lax.axis_index("s")
        wid = cid * NS + sid
        zeros16 = jnp.zeros((16,), jnp.float32)

        pltpu.sync_copy(col_hbm.at[wid], col2)
        pltpu.sync_copy(ew_hbm.at[wid], ew2)

        @pl.when(sid == 0)
        def _zero():
            def zr(i, _):
                zero_v[pl.ds(i * 16, 16)] = zeros16
                return 0
            lax.fori_loop(0, N_NODES // 16, zr, 0)
            pltpu.sync_copy(zero_v, acc_sh)

        plsc.subcore_barrier()

        def body(ci, _):
            pltpu.async_copy(ew2.at[ci], acc_sh.at[col2.at[ci]], sem, add=True)
            return 0
        lax.fori_loop(0, NCHUNK, body, 0)

        def drain(ci, _):
            pltpu.make_async_copy(ew2.at[0], acc_sh.at[col2.at[0]], sem).wait()
            return 0
        lax.fori_loop(0, NCHUNK, drain, 0)

        plsc.subcore_barrier()

        @pl.when(sid == 0)
        def _out():
            pltpu.sync_copy(acc_sh, out_hbm.at[cid])

    return deg_kernel


NBUF = 4


def _make_msg_kernel(F):
    @functools.partial(
        pl.kernel,
        out_type=jax.ShapeDtypeStruct((NC, N_NODES, F), jnp.float32),
        mesh=_mesh,
        scratch_types=[
            pltpu.VMEM((2 * NBUF, CH), jnp.int32),
            [pltpu.VMEM((CH,), jnp.int32)] * NBUF,
            [pltpu.VMEM((CH,), jnp.float32)] * NBUF,
            [pltpu.VMEM((CH, F), jnp.float32)] * NBUF,
            pltpu.VMEM_SHARED((N_NODES, F), jnp.float32),
            [pltpu.SemaphoreType.DMA] * NBUF,
            [pltpu.SemaphoreType.DMA] * NBUF,
            [pltpu.SemaphoreType.DMA] * NBUF,
        ],
    )
    def msg_kernel(xs_hbm, row_hbm, col_hbm, ew_hbm, out_hbm,
                   colb, rowb, ewb, msgs, acc_sh, isems, gsems, ssems):
        cid = lax.axis_index("c")
        sid = lax.axis_index("s")
        wid = cid * NS + sid
        ebase = wid * EPT
        zeros16 = jnp.zeros((16,), jnp.float32)
        zsem = isems[0]

        def zr(i, _):
            for f in range(F // 16):
                msgs[0][i, pl.ds(f * 16, 16)] = zeros16
            return 0
        lax.fori_loop(0, CH, zr, 0)

        rbase = sid * WB
        for b in range(WB // CH):
            pltpu.async_copy(msgs[0], acc_sh.at[pl.ds(rbase + b * CH, CH)],
                             zsem)
        zrem = WB - (WB // CH) * CH
        pltpu.async_copy(msgs[0].at[pl.ds(0, zrem)],
                         acc_sh.at[pl.ds(rbase + (WB // CH) * CH, zrem)], zsem)

        @pl.when(sid == 0)
        def _zrem():
            pltpu.async_copy(msgs[0].at[pl.ds(0, REM)],
                             acc_sh.at[pl.ds(NS * WB, REM)], zsem)

        for b in range(WB // CH):
            pltpu.make_async_copy(msgs[0],
                                  acc_sh.at[pl.ds(rbase + b * CH, CH)],
                                  zsem).wait()
        pltpu.make_async_copy(msgs[0].at[pl.ds(0, zrem)],
                              acc_sh.at[pl.ds(rbase, zrem)], zsem).wait()

        @pl.when(sid == 0)
        def _zremw():
            pltpu.make_async_copy(msgs[0].at[pl.ds(0, REM)],
                                  acc_sh.at[pl.ds(NS * WB, REM)], zsem).wait()

        plsc.subcore_barrier()

        def i_start(c, b):
            off = ebase + c * CH
            pltpu.async_copy(row_hbm.at[pl.ds(off, CH)], rowb[b], isems[b])
            pltpu.async_copy(ew_hbm.at[pl.ds(off, CH)], ewb[b], isems[b])
            pltpu.async_copy(col_hbm.at[wid, c],
                             colb.at[pl.ds(c % (2 * NBUF), 1)], isems[b])

        def i_wait(b):
            pltpu.make_async_copy(row_hbm.at[pl.ds(0, CH)], rowb[b],
                                  isems[b]).wait()
            pltpu.make_async_copy(ew_hbm.at[pl.ds(0, CH)], ewb[b],
                                  isems[b]).wait()
            pltpu.make_async_copy(col_hbm.at[0, 0],
                                  colb.at[pl.ds(0, 1)], isems[b]).wait()

        def g_start(b):
            pltpu.async_copy(xs_hbm.at[rowb[b]], msgs[b], gsems[b])

        def g_wait(b):
            pltpu.make_async_copy(xs_hbm.at[rowb[b]], msgs[b],
                                  gsems[b]).wait()

        def s_start(c, b):
            pltpu.async_copy(msgs[b], acc_sh.at[colb.at[c % (2 * NBUF)]],
                             ssems[b], add=True)

        def s_wait(b):
            pltpu.make_async_copy(msgs[b], acc_sh.at[colb.at[0]],
                                  ssems[b]).wait()

        def scale(c, b):
            def sc(j, _):
                w16 = ewb[b][pl.ds(j * 16, 16)]
                for l in range(16):
                    wl = w16[l]
                    e = j * 16 + l
                    for f in range(F // 16):
                        msgs[b][e, pl.ds(f * 16, 16)] = (
                            msgs[b][e, pl.ds(f * 16, 16)] * wl)
                return 0
            lax.fori_loop(0, CH // 16, sc, 0)

        def process(c, b):
            b2 = (b + 2) % NBUF
            b3 = (b + 3) % NBUF

            @pl.when(jnp.logical_and(c >= 2, c + 2 < NCHUNK))
            def _():
                s_wait(b2)

            @pl.when(c + 2 < NCHUNK)
            def _():
                i_wait(b2)
                g_start(b2)

            @pl.when(c + 3 < NCHUNK)
            def _():
                i_start(c + 3, b3)

            g_wait(b)
            scale(c, b)
            s_start(c, b)

        i_start(0, 0)
        i_start(1, 1)
        i_start(2, 2)
        i_wait(0)
        g_start(0)
        i_wait(1)
        g_start(1)

        NOUT = NCHUNK // NBUF

        def outer(i, _):
            for b in range(NBUF):
                process(i * NBUF + b, b)
            return 0
        lax.fori_loop(0, NOUT, outer, 0)
        for b in range(NCHUNK - NOUT * NBUF):
            process(NOUT * NBUF + b, b)
        for k in range(4):
            s_wait((NCHUNK - 4 + k) % NBUF)

        plsc.subcore_barrier()

        for b in range(WB // CH):
            pltpu.async_copy(acc_sh.at[pl.ds(rbase + b * CH, CH)],
                             out_hbm.at[cid, pl.ds(rbase + b * CH, CH)], zsem)
        pltpu.async_copy(acc_sh.at[pl.ds(rbase + (WB // CH) * CH, zrem)],
                         out_hbm.at[cid, pl.ds(rbase + (WB // CH) * CH, zrem)],
                         zsem)

        @pl.when(sid == 0)
        def _wrem():
            pltpu.async_copy(acc_sh.at[pl.ds(NS * WB, REM)],
                             out_hbm.at[cid, pl.ds(NS * WB, REM)], zsem)

        for b in range(WB // CH):
            pltpu.make_async_copy(acc_sh.at[pl.ds(rbase + b * CH, CH)],
                                  out_hbm.at[cid, pl.ds(rbase + b * CH, CH)],
                                  zsem).wait()
        pltpu.make_async_copy(acc_sh.at[pl.ds(rbase, zrem)],
                              out_hbm.at[cid, pl.ds(rbase, zrem)], zsem).wait()

        @pl.when(sid == 0)
        def _wremw():
            pltpu.make_async_copy(acc_sh.at[pl.ds(NS * WB, REM)],
                                  out_hbm.at[cid, pl.ds(NS * WB, REM)],
                                  zsem).wait()

    return msg_kernel


_deg_kernel = _make_deg_kernel()
_msg_kernel_128 = _make_msg_kernel(D)


def _mm_body(x_ref, w_ref, out_ref):
    out_ref[...] = jnp.dot(x_ref[...], w_ref[...],
                           preferred_element_type=jnp.float32)


def _prep_body(degp_ref, xw_ref, dinv_ref, xs_ref):
    deg = 1.0 + degp_ref[0] + degp_ref[1]
    dinv = jnp.where(deg > 0, lax.rsqrt(deg), 0.0)
    dinv_ref[...] = dinv
    xs_ref[...] = dinv * xw_ref[...]


def _combine_body(s_ref, xs_ref, dinv_ref, b_ref, w_ref, out_ref):
    s = s_ref[0] + s_ref[1] + xs_ref[...]
    h = jnp.maximum(dinv_ref[...] * s + b_ref[...], 0.0)
    out_ref[...] = dinv_ref[...] * jnp.dot(h, w_ref[...],
                                           preferred_element_type=jnp.float32)


def _final_body(s_ref, xs_ref, dinv_ref, b_ref, batch_ref, out_ref):
    o = dinv_ref[...] * (s_ref[0] + s_ref[1] + xs_ref[...]) + b_ref[...]
    gids = lax.broadcasted_iota(jnp.int32, (N_NODES, N_GRAPHS), 1)
    onehot = (batch_ref[...] == gids).astype(jnp.float32)
    sums = lax.dot_general(onehot, o, (((0,), (0,)), ((), ())))
    ones = jnp.ones((N_NODES, 1), jnp.float32)
    cnt = lax.dot_general(onehot, ones, (((0,), (0,)), ((), ())))
    mean = sums / jnp.maximum(cnt, 1.0)
    cols = lax.broadcasted_iota(jnp.int32, (N_GRAPHS, DP), 1)
    logits = jnp.where(cols < 10, mean, -1e30)
    m = jnp.max(logits, axis=1, keepdims=True)
    z = logits - m
    lse = jnp.log(jnp.sum(jnp.exp(z), axis=1, keepdims=True))
    out_ref[...] = z - lse


def _tc_call(body, out_shapes, *args):
    return pl.pallas_call(
        body,
        out_shape=out_shapes,
    )(*args)


def kernel(x, edge_index, edge_attr, batch, W1, b1, W2, b2, W3, b3):
    row = edge_index[0].astype(jnp.int32)
    col = edge_index[1].astype(jnp.int32)
    col3 = col.reshape(NW, NCHUNK, CH)
    col4 = col.reshape(NW, NCHUNK, 1, CH)
    ew = edge_attr.astype(jnp.float32)
    ew3 = ew.reshape(NW, NCHUNK, CH)
    batch2 = batch.astype(jnp.int32).reshape(N_NODES, 1)

    W3p = jnp.pad(W3, ((0, 0), (0, DP - W3.shape[1])))
    b3p = jnp.pad(b3, (0, DP - b3.shape[0])).reshape(1, DP)
    b1r = b1.reshape(1, D)
    b2r = b2.reshape(1, D)

    degp = _deg_kernel(col3, ew3)
    degp3 = degp.reshape(NC, N_NODES, 1)
    xw1 = _tc_call(_mm_body,
                   jax.ShapeDtypeStruct((N_NODES, D), jnp.float32),
                   x, W1)

    dinv, xs1 = _tc_call(
        _prep_body,
        [jax.ShapeDtypeStruct((N_NODES, 1), jnp.float32),
         jax.ShapeDtypeStruct((N_NODES, D), jnp.float32)],
        degp3, xw1)

    s1 = _msg_kernel_128(xs1, row, col4, ew)
    xs2 = _tc_call(
        _combine_body,
        jax.ShapeDtypeStruct((N_NODES, D), jnp.float32),
        s1, xs1, dinv, b1r, W2)

    s2 = _msg_kernel_128(xs2, row, col4, ew)
    xs3 = _tc_call(
        _combine_body,
        jax.ShapeDtypeStruct((N_NODES, DP), jnp.float32),
        s2, xs2, dinv, b2r, W3p)

    s3 = _msg_kernel_128(xs3, row, col4, ew)
    outp = _tc_call(
        _final_body,
        jax.ShapeDtypeStruct((N_GRAPHS, DP), jnp.float32),
        s3, xs3, dinv, b3p, batch2)

    return outp[:, :10]
